# Initial kernel scaffold; baseline (speedup 1.0000x reference)
import jax
import jax.numpy as jnp
from jax.experimental import pallas as pl


def kernel(x_protein, x_gene, ei_protein, ei_gene, params):
    raise NotImplementedError("write your pallas kernel here")


# speedup vs baseline: 12.8251x; 12.8251x over previous
import functools

import jax
import jax.numpy as jnp
from jax import lax
from jax.experimental import pallas as pl
from jax.experimental.pallas import tpu as pltpu
from jax.experimental.pallas import tpu_sc as plsc

EPS = 1e-5
_NC, _NS = 2, 16
_CH = 128


def _blocks(total, blk):
    out = []
    left = total
    while left > 0:
        b = min(blk, left)
        out.append((total - left, b))
        left -= b
    return out


def _sc_degree(dst_flat, n):
    e = dst_flat.shape[0] // 2
    ew = e // _NS
    n_full, tail = divmod(ew, _CH)
    mesh = plsc.VectorSubcoreMesh(core_axis_name="c", subcore_axis_name="s",
                                  num_cores=_NC, num_subcores=_NS)
    scratch = [
        pltpu.VMEM((_CH,), jnp.int32),
        pltpu.VMEM((_CH, 16), jnp.float32),
        pltpu.VMEM_SHARED((n, 16), jnp.float32),
    ]
    if tail:
        scratch.append(pltpu.VMEM((tail,), jnp.int32))
    init_ones = jnp.ones((n, 16), jnp.float32)
    ones_rows = jnp.ones((_CH, 16), jnp.float32)

    @functools.partial(
        pl.kernel,
        out_type=jax.ShapeDtypeStruct((2, n, 16), jnp.float32),
        mesh=mesh,
        scratch_types=scratch,
    )
    def deg_kernel(dst_hbm, init_hbm, ones_hbm, deg_out, idx_v, ones_v, acc,
                   *rest):
        c = lax.axis_index("c")
        s = lax.axis_index("s")

        @pl.when(s == 0)
        def _():
            pltpu.sync_copy(init_hbm, acc)
        pltpu.sync_copy(ones_hbm, ones_v)
        plsc.subcore_barrier()

        base = c * e + s * ew

        def step(i, carry):
            pltpu.sync_copy(dst_hbm.at[pl.ds(base + i * _CH, _CH)], idx_v)
            pltpu.sync_copy(ones_v, acc.at[idx_v], add=True)
            return carry
        lax.fori_loop(0, n_full, step, 0)
        if tail:
            idx_t = rest[0]
            pltpu.sync_copy(dst_hbm.at[pl.ds(base + n_full * _CH, tail)], idx_t)
            pltpu.sync_copy(ones_v.at[pl.ds(0, tail)], acc.at[idx_t], add=True)
        plsc.subcore_barrier()

        @pl.when(s == 0)
        def _():
            pltpu.sync_copy(acc, deg_out.at[c])

    return deg_kernel(dst_flat, init_ones, ones_rows)


def _sc_scatter(y_flat, src_flat, dst_flat, zeros_nc, n):
    e = src_flat.shape[0] // 2
    c_dim = y_flat.shape[1]
    ew = e // _NS
    n_full, tail = divmod(ew, _CH)
    mesh = plsc.VectorSubcoreMesh(core_axis_name="c", subcore_axis_name="s",
                                  num_cores=_NC, num_subcores=_NS)
    scratch = [
        pltpu.VMEM((_CH,), jnp.int32),
        pltpu.VMEM((_CH,), jnp.int32),
        pltpu.VMEM((_CH, c_dim), jnp.float32),
        pltpu.VMEM_SHARED((n, c_dim), jnp.float32),
        pltpu.SemaphoreType.DMA,
    ]
    if tail:
        scratch += [pltpu.VMEM((tail,), jnp.int32), pltpu.VMEM((tail,), jnp.int32)]

    @functools.partial(
        pl.kernel,
        out_type=jax.ShapeDtypeStruct((2, n, c_dim), jnp.float32),
        mesh=mesh,
        scratch_types=scratch,
    )
    def scat_kernel(y_hbm, src_hbm, dst_hbm, zeros_hbm, s_out,
                    idx_s, idx_d, rows_v, acc, sem, *rest):
        c = lax.axis_index("c")
        s = lax.axis_index("s")

        @pl.when(s == 0)
        def _():
            pltpu.sync_copy(zeros_hbm, acc)
        plsc.subcore_barrier()

        base = c * e + s * ew

        def step(i, carry):
            b = base + i * _CH
            pltpu.sync_copy(src_hbm.at[pl.ds(b, _CH)], idx_s)
            pltpu.sync_copy(dst_hbm.at[pl.ds(b, _CH)], idx_d)
            pltpu.async_copy(y_hbm.at[idx_s], rows_v, sem).wait()
            pltpu.sync_copy(rows_v, acc.at[idx_d], add=True)
            return carry
        lax.fori_loop(0, n_full, step, 0)
        if tail:
            idx_st, idx_dt = rest
            b = base + n_full * _CH
            pltpu.sync_copy(src_hbm.at[pl.ds(b, tail)], idx_st)
            pltpu.sync_copy(dst_hbm.at[pl.ds(b, tail)], idx_dt)
            pltpu.async_copy(y_hbm.at[idx_st], rows_v.at[pl.ds(0, tail)], sem).wait()
            pltpu.sync_copy(rows_v.at[pl.ds(0, tail)], acc.at[idx_dt], add=True)
        plsc.subcore_barrier()

        @pl.when(s == 0)
        def _():
            pltpu.sync_copy(acc, s_out.at[c])

    return scat_kernel(y_flat, src_flat, dst_flat, zeros_nc)


_BS = 1000


def _tc_pre(x_all, w_all, deg2d):
    two, n, cin = x_all.shape
    chid = w_all.shape[2]
    nb = n // _BS

    def body(x_ref, w_ref, d_ref, y_ref):
        dinv = lax.rsqrt(d_ref[0, :, 0:1])
        xw = jnp.dot(x_ref[0], w_ref[0], preferred_element_type=jnp.float32)
        y_ref[0] = dinv * xw

    return pl.pallas_call(
        body,
        grid=(two, nb),
        in_specs=[
            pl.BlockSpec((1, _BS, cin), lambda c, i: (c, i, 0)),
            pl.BlockSpec((1, cin, chid), lambda c, i: (c, 0, 0)),
            pl.BlockSpec((1, _BS, 16), lambda c, i: (c, i, 0)),
        ],
        out_specs=pl.BlockSpec((1, _BS, chid), lambda c, i: (c, i, 0)),
        out_shape=jax.ShapeDtypeStruct((two, n, chid), jnp.float32),
    )(x_all, w_all, deg2d)


def _tc_stats(s_all, y_all, deg2d, b_all):
    two, n, c_dim = s_all.shape
    nb = n // _BS

    def body(s_ref, y_ref, d_ref, b_ref, o_ref):
        i = pl.program_id(1)

        @pl.when(i == 0)
        def _():
            o_ref[...] = jnp.zeros_like(o_ref)

        dinv = lax.rsqrt(d_ref[0, :, 0:1])
        pre = dinv * (s_ref[0] + y_ref[0]) + b_ref[0]
        o_ref[0, 0:1, :] += jnp.sum(pre, axis=0, keepdims=True)
        o_ref[0, 1:2, :] += jnp.sum(pre * pre, axis=0, keepdims=True)

    return pl.pallas_call(
        body,
        grid=(two, nb),
        in_specs=[
            pl.BlockSpec((1, _BS, c_dim), lambda c, i: (c, i, 0)),
            pl.BlockSpec((1, _BS, c_dim), lambda c, i: (c, i, 0)),
            pl.BlockSpec((1, _BS, 16), lambda c, i: (c, i, 0)),
            pl.BlockSpec((1, 1, c_dim), lambda c, i: (c, 0, 0)),
        ],
        out_specs=pl.BlockSpec((1, 8, c_dim), lambda c, i: (c, 0, 0)),
        out_shape=jax.ShapeDtypeStruct((two, 8, c_dim), jnp.float32),
    )(s_all, y_all, deg2d, b_all)


def _gn_apply(pre, st_ref, gw_ref, gb_ref, gm_ref, n):
    s1 = st_ref[0, 0:1, :]
    s2 = st_ref[0, 1:2, :]
    mean = s1 * (1.0 / n)
    ms = gm_ref[0]
    var = s2 * (1.0 / n) - (2.0 - ms) * ms * mean * mean
    ctr = pre - ms * mean
    return ctr * lax.rsqrt(var + EPS) * gw_ref[0] + gb_ref[0]


def _tc_apply1(s_all, y_all, deg2d, stats, b_all, gnw, gnb, gnms, w2_all):
    two, n, c_dim = s_all.shape
    chid = w2_all.shape[2]
    nb = n // _BS

    def body(s_ref, y_ref, d_ref, st_ref, b_ref, gw_ref, gb_ref, gm_ref, w_ref,
             h1_ref, y2_ref):
        dinv = lax.rsqrt(d_ref[0, :, 0:1])
        pre = dinv * (s_ref[0] + y_ref[0]) + b_ref[0]
        h1 = jnp.maximum(_gn_apply(pre, st_ref, gw_ref, gb_ref, gm_ref, n), 0.0)
        h1_ref[0] = h1
        y2_ref[0] = dinv * jnp.dot(h1, w_ref[0], preferred_element_type=jnp.float32)

    return pl.pallas_call(
        body,
        grid=(two, nb),
        in_specs=[
            pl.BlockSpec((1, _BS, c_dim), lambda c, i: (c, i, 0)),
            pl.BlockSpec((1, _BS, c_dim), lambda c, i: (c, i, 0)),
            pl.BlockSpec((1, _BS, 16), lambda c, i: (c, i, 0)),
            pl.BlockSpec((1, 8, c_dim), lambda c, i: (c, 0, 0)),
            pl.BlockSpec((1, 1, c_dim), lambda c, i: (c, 0, 0)),
            pl.BlockSpec((1, 1, c_dim), lambda c, i: (c, 0, 0)),
            pl.BlockSpec((1, 1, c_dim), lambda c, i: (c, 0, 0)),
            pl.BlockSpec((1, 1, c_dim), lambda c, i: (c, 0, 0)),
            pl.BlockSpec((1, c_dim, chid), lambda c, i: (c, 0, 0)),
        ],
        out_specs=[
            pl.BlockSpec((1, _BS, c_dim), lambda c, i: (c, i, 0)),
            pl.BlockSpec((1, _BS, chid), lambda c, i: (c, i, 0)),
        ],
        out_shape=[
            jax.ShapeDtypeStruct((two, n, c_dim), jnp.float32),
            jax.ShapeDtypeStruct((two, n, chid), jnp.float32),
        ],
    )(s_all, y_all, deg2d, stats, b_all, gnw, gnb, gnms, w2_all)


def _tc_apply2(s_all, y2_all, deg2d, stats, b_all, gnw, gnb, gnms, h1_all):
    two, n, c_dim = s_all.shape
    nb = n // _BS

    def body(s_ref, y_ref, d_ref, st_ref, b_ref, gw_ref, gb_ref, gm_ref, h1_ref,
             o_ref):
        i = pl.program_id(1)
        dinv = lax.rsqrt(d_ref[0, :, 0:1])
        pre = dinv * (s_ref[0] + y_ref[0]) + b_ref[0]
        g = _gn_apply(pre, st_ref, gw_ref, gb_ref, gm_ref, n)
        h2 = jnp.maximum(g + h1_ref[0], 0.0)

        @pl.when(i == 0)
        def _():
            o_ref[...] = jnp.zeros_like(o_ref)

        o_ref[0, 0:1, :] += jnp.sum(h2, axis=0, keepdims=True)
        o_ref[0, 1:2, :] = jnp.maximum(o_ref[0, 1:2, :],
                                       jnp.max(h2, axis=0, keepdims=True))

    return pl.pallas_call(
        body,
        grid=(two, nb),
        in_specs=[
            pl.BlockSpec((1, _BS, c_dim), lambda c, i: (c, i, 0)),
            pl.BlockSpec((1, _BS, c_dim), lambda c, i: (c, i, 0)),
            pl.BlockSpec((1, _BS, 16), lambda c, i: (c, i, 0)),
            pl.BlockSpec((1, 8, c_dim), lambda c, i: (c, 0, 0)),
            pl.BlockSpec((1, 1, c_dim), lambda c, i: (c, 0, 0)),
            pl.BlockSpec((1, 1, c_dim), lambda c, i: (c, 0, 0)),
            pl.BlockSpec((1, 1, c_dim), lambda c, i: (c, 0, 0)),
            pl.BlockSpec((1, 1, c_dim), lambda c, i: (c, 0, 0)),
            pl.BlockSpec((1, _BS, c_dim), lambda c, i: (c, i, 0)),
        ],
        out_specs=pl.BlockSpec((1, 8, c_dim), lambda c, i: (c, 0, 0)),
        out_shape=jax.ShapeDtypeStruct((two, 8, c_dim), jnp.float32),
    )(s_all, y2_all, deg2d, stats, b_all, gnw, gnb, gnms, h1_all)


def _tc_heads(pool, n, w1_all, b1_all, lnw_all, lnb_all, w2_all, b2_all,
              att_w, att_b):
    c_out = w2_all.shape[2]

    def body(pool_ref, w1_ref, b1_ref, lnw_ref, lnb_ref, w2_ref, b2_ref,
             aw_ref, ab_ref, po_ref, go_ref, awo_ref, comb_ref):
        outs = []
        for r in range(2):
            f = jnp.concatenate(
                [pool_ref[r, 0:1, :] * (1.0 / n), pool_ref[r, 1:2, :]], axis=1)
            h = jnp.dot(f, w1_ref[r], preferred_element_type=jnp.float32) + b1_ref[r]
            m = jnp.mean(h, axis=-1, keepdims=True)
            v = jnp.mean((h - m) * (h - m), axis=-1, keepdims=True)
            h = (h - m) / jnp.sqrt(v + EPS) * lnw_ref[r] + lnb_ref[r]
            h = jnp.maximum(h, 0.0)
            outs.append(jnp.dot(h, w2_ref[r], preferred_element_type=jnp.float32)
                        + b2_ref[r])
        po, go = outs
        comb = jnp.concatenate([po, go], axis=1)
        logits = jnp.dot(comb, aw_ref[...], preferred_element_type=jnp.float32) \
            + ab_ref[...]
        emax = jnp.max(logits, axis=1, keepdims=True)
        ex = jnp.exp(logits - emax)
        aw = ex / jnp.sum(ex, axis=1, keepdims=True)
        po_ref[...] = po
        go_ref[...] = go
        awo_ref[...] = aw
        comb_ref[...] = aw[:, 0:1] * po + aw[:, 1:2] * go

    return pl.pallas_call(
        body,
        out_shape=[
            jax.ShapeDtypeStruct((1, c_out), jnp.float32),
            jax.ShapeDtypeStruct((1, c_out), jnp.float32),
            jax.ShapeDtypeStruct((1, 2), jnp.float32),
            jax.ShapeDtypeStruct((1, c_out), jnp.float32),
        ],
    )(pool, w1_all, b1_all, lnw_all, lnb_all, w2_all, b2_all, att_w, att_b)


def kernel(x_protein, x_gene, ei_protein, ei_gene, params):
    p = params
    n, cin = x_protein.shape

    x_all = jnp.stack([x_protein, x_gene])
    w1_all = jnp.stack([p['W1p'], p['W1g']])
    w2_all = jnp.stack([p['W2p'], p['W2g']])
    b1_all = jnp.stack([p['b1p'], p['b1g']])[:, None, :]
    b2_all = jnp.stack([p['b2p'], p['b2g']])[:, None, :]
    gn1w = jnp.stack([p['gn1p_w'], p['gn1g_w']])[:, None, :]
    gn1b = jnp.stack([p['gn1p_b'], p['gn1g_b']])[:, None, :]
    gn1m = jnp.stack([p['gn1p_ms'], p['gn1g_ms']])[:, None, :]
    gn2w = jnp.stack([p['gn2p_w'], p['gn2g_w']])[:, None, :]
    gn2b = jnp.stack([p['gn2p_b'], p['gn2g_b']])[:, None, :]
    gn2m = jnp.stack([p['gn2p_ms'], p['gn2g_ms']])[:, None, :]
    hw1 = jnp.stack([p['hp_W1'], p['hg_W1']])
    hb1 = jnp.stack([p['hp_b1'], p['hg_b1']])[:, None, :]
    hlnw = jnp.stack([p['hp_lnw'], p['hg_lnw']])[:, None, :]
    hlnb = jnp.stack([p['hp_lnb'], p['hg_lnb']])[:, None, :]
    hw2 = jnp.stack([p['hp_W2'], p['hg_W2']])
    hb2 = jnp.stack([p['hp_b2'], p['hg_b2']])[:, None, :]
    att_w = p['att_W']
    att_b = p['att_b'][None, :]

    dst_all = jnp.concatenate([ei_protein[1], ei_gene[1]])
    src_all = jnp.concatenate([ei_protein[0], ei_gene[0] + n])

    zeros_nc = jnp.zeros((n, cin), jnp.float32)

    deg2d = _sc_degree(dst_all, n)
    y1 = _tc_pre(x_all, w1_all, deg2d)
    s1 = _sc_scatter(y1.reshape(2 * n, -1), src_all, dst_all, zeros_nc, n)
    st1 = _tc_stats(s1, y1, deg2d, b1_all)
    h1, y2 = _tc_apply1(s1, y1, deg2d, st1, b1_all, gn1w, gn1b, gn1m, w2_all)
    s2 = _sc_scatter(y2.reshape(2 * n, -1), src_all, dst_all, zeros_nc, n)
    st2 = _tc_stats(s2, y2, deg2d, b2_all)
    pool = _tc_apply2(s2, y2, deg2d, st2, b2_all, gn2w, gn2b, gn2m, h1)
    po, go, aw, comb = _tc_heads(pool, n, hw1, hb1, hlnw, hlnb, hw2, hb2,
                                 att_w, att_b)
    return po, go, aw, comb


# speedup vs baseline: 22.3964x; 1.7463x over previous
import functools

import jax
import jax.numpy as jnp
from jax import lax
from jax.experimental import pallas as pl
from jax.experimental.pallas import tpu as pltpu
from jax.experimental.pallas import tpu_sc as plsc

EPS = 1e-5
_NC, _NS = 2, 16
_CH = 128


def _blocks(total, blk):
    out = []
    left = total
    while left > 0:
        b = min(blk, left)
        out.append((total - left, b))
        left -= b
    return out


_KB = 8
_JUNK = 128


def _sc_degree(dst_flat, n):
    ep = dst_flat.shape[0] // 2
    ew = ep // _NS
    nchunk = ew // _CH
    nn = n + _JUNK
    mesh = plsc.VectorSubcoreMesh(core_axis_name="c", subcore_axis_name="s",
                                  num_cores=_NC, num_subcores=_NS)
    scratch = [
        pltpu.VMEM((_CH,), jnp.int32),
        pltpu.VMEM((_CH,), jnp.int32),
        pltpu.VMEM((_CH, 16), jnp.float32),
        pltpu.VMEM_SHARED((nn, 16), jnp.float32),
        pltpu.SemaphoreType.DMA,
        pltpu.SemaphoreType.DMA,
    ]
    init_ones = jnp.ones((nn, 16), jnp.float32)
    ones_rows = jnp.ones((_CH, 16), jnp.float32)

    @functools.partial(
        pl.kernel,
        out_type=jax.ShapeDtypeStruct((2, nn, 16), jnp.float32),
        mesh=mesh,
        scratch_types=scratch,
    )
    def deg_kernel(dst_hbm, init_hbm, ones_hbm, deg_out, idx_a, idx_b, ones_v,
                   acc, isem_a, isem_b):
        c = lax.axis_index("c")
        s = lax.axis_index("s")

        @pl.when(s == 0)
        def _():
            pltpu.sync_copy(init_hbm, acc)
        pltpu.sync_copy(ones_hbm, ones_v)
        plsc.subcore_barrier()

        base = c * ep + s * ew
        nb2 = nchunk // 2
        ibufs = [(idx_a, isem_a), (idx_b, isem_b)]

        pltpu.async_copy(dst_hbm.at[pl.ds(base, _CH)], idx_a, isem_a)

        def step(j, carry):
            e0 = base + j * 2 * _CH
            for k in range(2):
                iv, ism = ibufs[k]
                niv, nism = ibufs[(k + 1) % 2]
                pltpu.async_copy(
                    dst_hbm.at[pl.ds(e0 + (k + 1) * _CH, _CH)], niv, nism)
                pltpu.make_async_copy(
                    dst_hbm.at[pl.ds(e0 + k * _CH, _CH)], iv, ism).wait()
                pltpu.sync_copy(ones_v, acc.at[iv], add=True)
            return carry
        lax.fori_loop(0, nb2 - 1, step, 0)
        e0 = base + (nchunk - 2) * _CH
        pltpu.make_async_copy(dst_hbm.at[pl.ds(e0, _CH)], idx_a, isem_a).wait()
        pltpu.async_copy(dst_hbm.at[pl.ds(e0 + _CH, _CH)], idx_b, isem_b)
        pltpu.sync_copy(ones_v, acc.at[idx_a], add=True)
        pltpu.make_async_copy(
            dst_hbm.at[pl.ds(e0 + _CH, _CH)], idx_b, isem_b).wait()
        pltpu.sync_copy(ones_v, acc.at[idx_b], add=True)
        plsc.subcore_barrier()

        @pl.when(s == 0)
        def _():
            pltpu.sync_copy(acc, deg_out.at[c])

    return deg_kernel(dst_flat, init_ones, ones_rows)


def _sc_scatter(y_flat, src2d, dst_flat, zeros_nc, n):
    rows_total = src2d.shape[0] // 2
    c_dim = y_flat.shape[1]
    rpt = rows_total // _NS
    nb = rpt // _KB
    ep = rows_total * _CH
    ew = rpt * _CH
    nn = n + _JUNK
    mesh = plsc.VectorSubcoreMesh(core_axis_name="c", subcore_axis_name="s",
                                  num_cores=_NC, num_subcores=_NS)
    scratch = [
        pltpu.VMEM((_KB, _CH), jnp.int32),
        pltpu.VMEM((_CH,), jnp.int32),
        pltpu.VMEM((_CH,), jnp.int32),
        pltpu.VMEM((_CH, c_dim), jnp.float32),
        pltpu.VMEM((_CH, c_dim), jnp.float32),
        pltpu.VMEM_SHARED((nn, c_dim), jnp.float32),
        pltpu.SemaphoreType.DMA,
        pltpu.SemaphoreType.DMA,
        pltpu.SemaphoreType.DMA,
        pltpu.SemaphoreType.DMA,
    ]

    @functools.partial(
        pl.kernel,
        out_type=jax.ShapeDtypeStruct((2, nn, c_dim), jnp.float32),
        mesh=mesh,
        scratch_types=scratch,
    )
    def scat_kernel(y_hbm, src_hbm, dst_hbm, zeros_hbm, s_out,
                    idx_s, idx_wa, idx_wb, rows_a, rows_b, acc,
                    sem_a, sem_b, isem_a, isem_b):
        c = lax.axis_index("c")
        s = lax.axis_index("s")

        @pl.when(s == 0)
        def _():
            pltpu.sync_copy(zeros_hbm, acc)
        plsc.subcore_barrier()

        base_row = c * rows_total + s * rpt
        bufs = [(rows_a, sem_a), (rows_b, sem_b)]
        base_e = c * ep + s * ew
        ibufs = [(idx_wa, isem_a), (idx_wb, isem_b)]

        def block(j, carry):
            r0 = base_row + j * _KB
            e0 = base_e + j * _KB * _CH
            pltpu.sync_copy(src_hbm.at[pl.ds(r0, _KB)], idx_s)
            pltpu.async_copy(y_hbm.at[idx_s.at[0]], rows_a, sem_a)
            pltpu.async_copy(dst_hbm.at[pl.ds(e0, _CH)], idx_wa, isem_a)
            for k in range(_KB):
                rv, sm = bufs[k % 2]
                iv, ism = ibufs[k % 2]
                if k + 1 < _KB:
                    nrv, nsm = bufs[(k + 1) % 2]
                    niv, nism = ibufs[(k + 1) % 2]
                    pltpu.async_copy(y_hbm.at[idx_s.at[k + 1]], nrv, nsm)
                    pltpu.async_copy(
                        dst_hbm.at[pl.ds(e0 + (k + 1) * _CH, _CH)], niv, nism)
                pltpu.make_async_copy(
                    dst_hbm.at[pl.ds(e0 + k * _CH, _CH)], iv, ism).wait()
                pltpu.make_async_copy(y_hbm.at[idx_s.at[k]], rv, sm).wait()
                pltpu.sync_copy(rv, acc.at[iv], add=True)
            return carry
        lax.fori_loop(0, nb, block, 0)
        plsc.subcore_barrier()

        @pl.when(s == 0)
        def _():
            pltpu.sync_copy(acc, s_out.at[c])

    return scat_kernel(y_flat, src2d, dst_flat, zeros_nc)


_BS = 1000


def _tc_pre(x_all, w_all, deg2d):
    two, n, cin = x_all.shape
    chid = w_all.shape[2]
    nb = n // _BS

    def body(x_ref, w_ref, d_ref, y_ref):
        dinv = lax.rsqrt(d_ref[0, :, 0:1])
        xw = jnp.dot(x_ref[0], w_ref[0], preferred_element_type=jnp.float32)
        y_ref[0] = dinv * xw

    return pl.pallas_call(
        body,
        grid=(two, nb),
        in_specs=[
            pl.BlockSpec((1, _BS, cin), lambda c, i: (c, i, 0)),
            pl.BlockSpec((1, cin, chid), lambda c, i: (c, 0, 0)),
            pl.BlockSpec((1, _BS, 16), lambda c, i: (c, i, 0)),
        ],
        out_specs=pl.BlockSpec((1, _BS, chid), lambda c, i: (c, i, 0)),
        out_shape=jax.ShapeDtypeStruct((two, n, chid), jnp.float32),
    )(x_all, w_all, deg2d)


def _gn_apply(pre, st, gw_ref, gb_ref, gm_ref, n):
    s1 = st[0:1, :]
    s2 = st[1:2, :]
    mean = s1 * (1.0 / n)
    ms = gm_ref[0]
    var = s2 * (1.0 / n) - (2.0 - ms) * ms * mean * mean
    ctr = pre - ms * mean
    return ctr * lax.rsqrt(var + EPS) * gw_ref[0] + gb_ref[0]


def _tc_layer1(s_all, y_all, deg2d, b_all, gnw, gnb, gnms, w2_all):
    two, n, c_dim = y_all.shape
    chid = w2_all.shape[2]
    nb = n // _BS

    def body(s_ref, y_ref, d_ref, b_ref, gw_ref, gb_ref, gm_ref, w_ref,
             h1_ref, y2_ref, st_ref):
        p = pl.program_id(1)
        i = pl.program_id(2)
        dinv = lax.rsqrt(d_ref[0, :, 0:1])
        pre = dinv * (s_ref[0] + y_ref[0]) + b_ref[0]

        @pl.when((p == 0) & (i == 0))
        def _():
            st_ref[...] = jnp.zeros_like(st_ref)

        @pl.when(p == 0)
        def _():
            st_ref[0:1, :] += jnp.sum(pre, axis=0, keepdims=True)
            st_ref[1:2, :] += jnp.sum(pre * pre, axis=0, keepdims=True)

        @pl.when(p == 1)
        def _():
            h1 = jnp.maximum(
                _gn_apply(pre, st_ref, gw_ref, gb_ref, gm_ref, n), 0.0)
            h1_ref[0] = h1
            y2_ref[0] = dinv * jnp.dot(h1, w_ref[0],
                                       preferred_element_type=jnp.float32)

    def out_map(c, p, i):
        return (c, jnp.where(p == 1, i, 0), 0)

    return pl.pallas_call(
        body,
        grid=(two, 2, nb),
        in_specs=[
            pl.BlockSpec((1, _BS, c_dim), lambda c, p, i: (c, i, 0)),
            pl.BlockSpec((1, _BS, c_dim), lambda c, p, i: (c, i, 0)),
            pl.BlockSpec((1, _BS, 16), lambda c, p, i: (c, i, 0)),
            pl.BlockSpec((1, 1, c_dim), lambda c, p, i: (c, 0, 0)),
            pl.BlockSpec((1, 1, c_dim), lambda c, p, i: (c, 0, 0)),
            pl.BlockSpec((1, 1, c_dim), lambda c, p, i: (c, 0, 0)),
            pl.BlockSpec((1, 1, c_dim), lambda c, p, i: (c, 0, 0)),
            pl.BlockSpec((1, c_dim, chid), lambda c, p, i: (c, 0, 0)),
        ],
        out_specs=[
            pl.BlockSpec((1, _BS, c_dim), out_map),
            pl.BlockSpec((1, _BS, chid), out_map),
        ],
        out_shape=[
            jax.ShapeDtypeStruct((two, n, c_dim), jnp.float32),
            jax.ShapeDtypeStruct((two, n, chid), jnp.float32),
        ],
        scratch_shapes=[pltpu.VMEM((8, c_dim), jnp.float32)],
    )(s_all, y_all, deg2d, b_all, gnw, gnb, gnms, w2_all)


def _tc_layer2(s_all, y2_all, deg2d, b_all, gnw, gnb, gnms, h1_all):
    two, n, c_dim = y2_all.shape
    nb = n // _BS

    def body(s_ref, y_ref, d_ref, b_ref, gw_ref, gb_ref, gm_ref, h1_ref,
             o_ref, st_ref):
        p = pl.program_id(1)
        i = pl.program_id(2)
        dinv = lax.rsqrt(d_ref[0, :, 0:1])
        pre = dinv * (s_ref[0] + y_ref[0]) + b_ref[0]

        @pl.when((p == 0) & (i == 0))
        def _():
            st_ref[...] = jnp.zeros_like(st_ref)

        @pl.when(p == 0)
        def _():
            st_ref[0:1, :] += jnp.sum(pre, axis=0, keepdims=True)
            st_ref[1:2, :] += jnp.sum(pre * pre, axis=0, keepdims=True)

        @pl.when(p == 1)
        def _():
            g = _gn_apply(pre, st_ref, gw_ref, gb_ref, gm_ref, n)
            h2 = jnp.maximum(g + h1_ref[0], 0.0)

            @pl.when(i == 0)
            def _():
                o_ref[...] = jnp.zeros_like(o_ref)

            o_ref[0, 0:1, :] += jnp.sum(h2, axis=0, keepdims=True)
            o_ref[0, 1:2, :] = jnp.maximum(o_ref[0, 1:2, :],
                                           jnp.max(h2, axis=0, keepdims=True))

    return pl.pallas_call(
        body,
        grid=(two, 2, nb),
        in_specs=[
            pl.BlockSpec((1, _BS, c_dim), lambda c, p, i: (c, i, 0)),
            pl.BlockSpec((1, _BS, c_dim), lambda c, p, i: (c, i, 0)),
            pl.BlockSpec((1, _BS, 16), lambda c, p, i: (c, i, 0)),
            pl.BlockSpec((1, 1, c_dim), lambda c, p, i: (c, 0, 0)),
            pl.BlockSpec((1, 1, c_dim), lambda c, p, i: (c, 0, 0)),
            pl.BlockSpec((1, 1, c_dim), lambda c, p, i: (c, 0, 0)),
            pl.BlockSpec((1, 1, c_dim), lambda c, p, i: (c, 0, 0)),
            pl.BlockSpec((1, _BS, c_dim), lambda c, p, i: (c, i, 0)),
        ],
        out_specs=pl.BlockSpec((1, 8, c_dim), lambda c, p, i: (c, 0, 0)),
        out_shape=jax.ShapeDtypeStruct((two, 8, c_dim), jnp.float32),
        scratch_shapes=[pltpu.VMEM((8, c_dim), jnp.float32)],
    )(s_all, y2_all, deg2d, b_all, gnw, gnb, gnms, h1_all)


def _tc_heads(pool, n, w1_all, b1_all, lnw_all, lnb_all, w2_all, b2_all,
              att_w, att_b):
    c_out = w2_all.shape[2]

    def body(pool_ref, w1_ref, b1_ref, lnw_ref, lnb_ref, w2_ref, b2_ref,
             aw_ref, ab_ref, po_ref, go_ref, awo_ref, comb_ref):
        outs = []
        for r in range(2):
            f = jnp.concatenate(
                [pool_ref[r, 0:1, :] * (1.0 / n), pool_ref[r, 1:2, :]], axis=1)
            h = jnp.dot(f, w1_ref[r], preferred_element_type=jnp.float32) + b1_ref[r]
            m = jnp.mean(h, axis=-1, keepdims=True)
            v = jnp.mean((h - m) * (h - m), axis=-1, keepdims=True)
            h = (h - m) / jnp.sqrt(v + EPS) * lnw_ref[r] + lnb_ref[r]
            h = jnp.maximum(h, 0.0)
            outs.append(jnp.dot(h, w2_ref[r], preferred_element_type=jnp.float32)
                        + b2_ref[r])
        po, go = outs
        comb = jnp.concatenate([po, go], axis=1)
        logits = jnp.dot(comb, aw_ref[...], preferred_element_type=jnp.float32) \
            + ab_ref[...]
        emax = jnp.max(logits, axis=1, keepdims=True)
        ex = jnp.exp(logits - emax)
        aw = ex / jnp.sum(ex, axis=1, keepdims=True)
        po_ref[...] = po
        go_ref[...] = go
        awo_ref[...] = aw
        comb_ref[...] = aw[:, 0:1] * po + aw[:, 1:2] * go

    return pl.pallas_call(
        body,
        out_shape=[
            jax.ShapeDtypeStruct((1, c_out), jnp.float32),
            jax.ShapeDtypeStruct((1, c_out), jnp.float32),
            jax.ShapeDtypeStruct((1, 2), jnp.float32),
            jax.ShapeDtypeStruct((1, c_out), jnp.float32),
        ],
    )(pool, w1_all, b1_all, lnw_all, lnb_all, w2_all, b2_all, att_w, att_b)


def kernel(x_protein, x_gene, ei_protein, ei_gene, params):
    p = params
    n, cin = x_protein.shape

    x_all = jnp.stack([x_protein, x_gene])
    w1_all = jnp.stack([p['W1p'], p['W1g']])
    w2_all = jnp.stack([p['W2p'], p['W2g']])
    b1_all = jnp.stack([p['b1p'], p['b1g']])[:, None, :]
    b2_all = jnp.stack([p['b2p'], p['b2g']])[:, None, :]
    gn1w = jnp.stack([p['gn1p_w'], p['gn1g_w']])[:, None, :]
    gn1b = jnp.stack([p['gn1p_b'], p['gn1g_b']])[:, None, :]
    gn1m = jnp.stack([p['gn1p_ms'], p['gn1g_ms']])[:, None, :]
    gn2w = jnp.stack([p['gn2p_w'], p['gn2g_w']])[:, None, :]
    gn2b = jnp.stack([p['gn2p_b'], p['gn2g_b']])[:, None, :]
    gn2m = jnp.stack([p['gn2p_ms'], p['gn2g_ms']])[:, None, :]
    hw1 = jnp.stack([p['hp_W1'], p['hg_W1']])
    hb1 = jnp.stack([p['hp_b1'], p['hg_b1']])[:, None, :]
    hlnw = jnp.stack([p['hp_lnw'], p['hg_lnw']])[:, None, :]
    hlnb = jnp.stack([p['hp_lnb'], p['hg_lnb']])[:, None, :]
    hw2 = jnp.stack([p['hp_W2'], p['hg_W2']])
    hb2 = jnp.stack([p['hp_b2'], p['hg_b2']])[:, None, :]
    att_w = p['att_W']
    att_b = p['att_b'][None, :]

    e = ei_protein.shape[1]
    align = _NS * _CH * 8
    ep = -(-e // align) * align
    pad = ep - e
    pad_dst = n + (jnp.arange(pad, dtype=jnp.int32) % _JUNK)
    pad_src = jnp.arange(pad, dtype=jnp.int32) % n
    dst_all = jnp.concatenate([ei_protein[1], pad_dst, ei_gene[1], pad_dst])
    src_all = jnp.concatenate(
        [ei_protein[0], pad_src, ei_gene[0] + n, pad_src]).reshape(-1, _CH)

    zeros_nc = jnp.zeros((n + _JUNK, cin), jnp.float32)

    deg2d = _sc_degree(dst_all, n)
    y1 = _tc_pre(x_all, w1_all, deg2d)
    s1 = _sc_scatter(y1.reshape(2 * n, -1), src_all, dst_all, zeros_nc, n)
    h1, y2 = _tc_layer1(s1, y1, deg2d, b1_all, gn1w, gn1b, gn1m, w2_all)
    s2 = _sc_scatter(y2.reshape(2 * n, -1), src_all, dst_all, zeros_nc, n)
    pool = _tc_layer2(s2, y2, deg2d, b2_all, gn2w, gn2b, gn2m, h1)
    po, go, aw, comb = _tc_heads(pool, n, hw1, hb1, hlnw, hlnb, hw2, hb2,
                                 att_w, att_b)
    return po, go, aw, comb


# speedup vs baseline: 22.6146x; 1.0097x over previous
import functools

import jax
import jax.numpy as jnp
from jax import lax
from jax.experimental import pallas as pl
from jax.experimental.pallas import tpu as pltpu
from jax.experimental.pallas import tpu_sc as plsc

EPS = 1e-5
_NC, _NS = 2, 16
_CH = 128


def _blocks(total, blk):
    out = []
    left = total
    while left > 0:
        b = min(blk, left)
        out.append((total - left, b))
        left -= b
    return out


_KB = 8
_JUNK = 128


def _sc_degree(dst_flat, n):
    ep = dst_flat.shape[0] // 2
    ew = ep // _NS
    nchunk = ew // _CH
    nn = n + _JUNK
    mesh = plsc.VectorSubcoreMesh(core_axis_name="c", subcore_axis_name="s",
                                  num_cores=_NC, num_subcores=_NS)
    scratch = [
        pltpu.VMEM((_CH,), jnp.int32),
        pltpu.VMEM((_CH,), jnp.int32),
        pltpu.VMEM((_CH, 16), jnp.float32),
        pltpu.VMEM_SHARED((nn, 16), jnp.float32),
        pltpu.SemaphoreType.DMA,
        pltpu.SemaphoreType.DMA,
    ]
    init_ones = jnp.ones((nn, 16), jnp.float32)
    ones_rows = jnp.ones((_CH, 16), jnp.float32)

    @functools.partial(
        pl.kernel,
        out_type=jax.ShapeDtypeStruct((2, nn, 16), jnp.float32),
        mesh=mesh,
        scratch_types=scratch,
    )
    def deg_kernel(dst_hbm, init_hbm, ones_hbm, deg_out, idx_a, idx_b, ones_v,
                   acc, isem_a, isem_b):
        c = lax.axis_index("c")
        s = lax.axis_index("s")

        @pl.when(s == 0)
        def _():
            pltpu.sync_copy(init_hbm, acc)
        pltpu.sync_copy(ones_hbm, ones_v)
        plsc.subcore_barrier()

        base = c * ep + s * ew
        nb2 = nchunk // 2
        ibufs = [(idx_a, isem_a), (idx_b, isem_b)]

        pltpu.async_copy(dst_hbm.at[pl.ds(base, _CH)], idx_a, isem_a)

        def step(j, carry):
            e0 = base + j * 2 * _CH
            for k in range(2):
                iv, ism = ibufs[k]
                niv, nism = ibufs[(k + 1) % 2]
                pltpu.async_copy(
                    dst_hbm.at[pl.ds(e0 + (k + 1) * _CH, _CH)], niv, nism)
                pltpu.make_async_copy(
                    dst_hbm.at[pl.ds(e0 + k * _CH, _CH)], iv, ism).wait()
                pltpu.sync_copy(ones_v, acc.at[iv], add=True)
            return carry
        lax.fori_loop(0, nb2 - 1, step, 0)
        e0 = base + (nchunk - 2) * _CH
        pltpu.make_async_copy(dst_hbm.at[pl.ds(e0, _CH)], idx_a, isem_a).wait()
        pltpu.async_copy(dst_hbm.at[pl.ds(e0 + _CH, _CH)], idx_b, isem_b)
        pltpu.sync_copy(ones_v, acc.at[idx_a], add=True)
        pltpu.make_async_copy(
            dst_hbm.at[pl.ds(e0 + _CH, _CH)], idx_b, isem_b).wait()
        pltpu.sync_copy(ones_v, acc.at[idx_b], add=True)
        plsc.subcore_barrier()

        @pl.when(s == 0)
        def _():
            pltpu.sync_copy(acc, deg_out.at[c])

    return deg_kernel(dst_flat, init_ones, ones_rows)


def _sc_scatter(y_flat, src2d, dst_flat, zeros_nc, n):
    rows_total = src2d.shape[0] // 2
    c_dim = y_flat.shape[1]
    rpt = rows_total // _NS
    nb = rpt // _KB
    ep = rows_total * _CH
    ew = rpt * _CH
    nn = n + _JUNK
    mesh = plsc.VectorSubcoreMesh(core_axis_name="c", subcore_axis_name="s",
                                  num_cores=_NC, num_subcores=_NS)
    scratch = [
        pltpu.VMEM((_KB, _CH), jnp.int32),
        pltpu.VMEM((_CH,), jnp.int32),
        pltpu.VMEM((_CH,), jnp.int32),
        pltpu.VMEM((_CH, c_dim), jnp.float32),
        pltpu.VMEM((_CH, c_dim), jnp.float32),
        pltpu.VMEM_SHARED((nn, c_dim), jnp.float32),
        pltpu.SemaphoreType.DMA,
        pltpu.SemaphoreType.DMA,
        pltpu.SemaphoreType.DMA,
        pltpu.SemaphoreType.DMA,
    ]

    @functools.partial(
        pl.kernel,
        out_type=jax.ShapeDtypeStruct((2, nn, c_dim), jnp.float32),
        mesh=mesh,
        scratch_types=scratch,
    )
    def scat_kernel(y_hbm, src_hbm, dst_hbm, zeros_hbm, s_out,
                    idx_s, idx_wa, idx_wb, rows_a, rows_b, acc,
                    sem_a, sem_b, isem_a, isem_b):
        c = lax.axis_index("c")
        s = lax.axis_index("s")

        @pl.when(s == 0)
        def _():
            pltpu.sync_copy(zeros_hbm, acc)
        plsc.subcore_barrier()

        base_row = c * rows_total + s * rpt
        bufs = [(rows_a, sem_a), (rows_b, sem_b)]
        base_e = c * ep + s * ew
        ibufs = [(idx_wa, isem_a), (idx_wb, isem_b)]

        def block(j, carry):
            r0 = base_row + j * _KB
            e0 = base_e + j * _KB * _CH
            pltpu.sync_copy(src_hbm.at[pl.ds(r0, _KB)], idx_s)
            pltpu.async_copy(y_hbm.at[idx_s.at[0]], rows_a, sem_a)
            pltpu.async_copy(dst_hbm.at[pl.ds(e0, _CH)], idx_wa, isem_a)
            for k in range(_KB):
                rv, sm = bufs[k % 2]
                iv, ism = ibufs[k % 2]
                if k + 1 < _KB:
                    nrv, nsm = bufs[(k + 1) % 2]
                    niv, nism = ibufs[(k + 1) % 2]
                    pltpu.async_copy(y_hbm.at[idx_s.at[k + 1]], nrv, nsm)
                    pltpu.async_copy(
                        dst_hbm.at[pl.ds(e0 + (k + 1) * _CH, _CH)], niv, nism)
                pltpu.make_async_copy(
                    dst_hbm.at[pl.ds(e0 + k * _CH, _CH)], iv, ism).wait()
                pltpu.make_async_copy(y_hbm.at[idx_s.at[k]], rv, sm).wait()
                pltpu.sync_copy(rv, acc.at[iv], add=True)
            return carry
        lax.fori_loop(0, nb, block, 0)
        plsc.subcore_barrier()

        @pl.when(s == 0)
        def _():
            pltpu.sync_copy(acc, s_out.at[c])

    return scat_kernel(y_flat, src2d, dst_flat, zeros_nc)


_BS = 1000


def _tc_pre(x_all, w_all, deg2d):
    two, n, cin = x_all.shape
    chid = w_all.shape[2]
    nb = n // _BS

    def body(x_ref, w_ref, d_ref, y_ref):
        dinv = lax.rsqrt(d_ref[0, :, 0:1])
        xw = jnp.dot(x_ref[0], w_ref[0], preferred_element_type=jnp.float32)
        y_ref[0] = dinv * xw

    return pl.pallas_call(
        body,
        grid=(two, nb),
        in_specs=[
            pl.BlockSpec((1, _BS, cin), lambda c, i: (c, i, 0)),
            pl.BlockSpec((1, cin, chid), lambda c, i: (c, 0, 0)),
            pl.BlockSpec((1, _BS, 16), lambda c, i: (c, i, 0)),
        ],
        out_specs=pl.BlockSpec((1, _BS, chid), lambda c, i: (c, i, 0)),
        out_shape=jax.ShapeDtypeStruct((two, n, chid), jnp.float32),
    )(x_all, w_all, deg2d)


def _tc_stats(s_all, y_all, deg2d, b_all):
    two, n, c_dim = y_all.shape
    nb = n // _BS

    def body(s_ref, y_ref, d_ref, b_ref, o_ref):
        i = pl.program_id(1)

        @pl.when(i == 0)
        def _():
            o_ref[...] = jnp.zeros_like(o_ref)

        dinv = lax.rsqrt(d_ref[0, :, 0:1])
        pre = dinv * (s_ref[0] + y_ref[0]) + b_ref[0]
        o_ref[0, 0:1, :] += jnp.sum(pre, axis=0, keepdims=True)
        o_ref[0, 1:2, :] += jnp.sum(pre * pre, axis=0, keepdims=True)

    return pl.pallas_call(
        body,
        grid=(two, nb),
        in_specs=[
            pl.BlockSpec((1, _BS, c_dim), lambda c, i: (c, i, 0)),
            pl.BlockSpec((1, _BS, c_dim), lambda c, i: (c, i, 0)),
            pl.BlockSpec((1, _BS, 16), lambda c, i: (c, i, 0)),
            pl.BlockSpec((1, 1, c_dim), lambda c, i: (c, 0, 0)),
        ],
        out_specs=pl.BlockSpec((1, 8, c_dim), lambda c, i: (c, 0, 0)),
        out_shape=jax.ShapeDtypeStruct((two, 8, c_dim), jnp.float32),
    )(s_all, y_all, deg2d, b_all)


def _gn_apply(pre, st_ref, gw_ref, gb_ref, gm_ref, n):
    s1 = st_ref[0, 0:1, :]
    s2 = st_ref[0, 1:2, :]
    mean = s1 * (1.0 / n)
    ms = gm_ref[0]
    var = s2 * (1.0 / n) - (2.0 - ms) * ms * mean * mean
    ctr = pre - ms * mean
    return ctr * lax.rsqrt(var + EPS) * gw_ref[0] + gb_ref[0]


def _tc_apply1(s_all, y_all, deg2d, stats, b_all, gnw, gnb, gnms, w2_all):
    two, n, c_dim = y_all.shape
    chid = w2_all.shape[2]
    nb = n // _BS

    def body(s_ref, y_ref, d_ref, st_ref, b_ref, gw_ref, gb_ref, gm_ref, w_ref,
             h1_ref, y2_ref):
        dinv = lax.rsqrt(d_ref[0, :, 0:1])
        pre = dinv * (s_ref[0] + y_ref[0]) + b_ref[0]
        h1 = jnp.maximum(_gn_apply(pre, st_ref, gw_ref, gb_ref, gm_ref, n), 0.0)
        h1_ref[0] = h1
        y2_ref[0] = dinv * jnp.dot(h1, w_ref[0], preferred_element_type=jnp.float32)

    return pl.pallas_call(
        body,
        grid=(two, nb),
        in_specs=[
            pl.BlockSpec((1, _BS, c_dim), lambda c, i: (c, i, 0)),
            pl.BlockSpec((1, _BS, c_dim), lambda c, i: (c, i, 0)),
            pl.BlockSpec((1, _BS, 16), lambda c, i: (c, i, 0)),
            pl.BlockSpec((1, 8, c_dim), lambda c, i: (c, 0, 0)),
            pl.BlockSpec((1, 1, c_dim), lambda c, i: (c, 0, 0)),
            pl.BlockSpec((1, 1, c_dim), lambda c, i: (c, 0, 0)),
            pl.BlockSpec((1, 1, c_dim), lambda c, i: (c, 0, 0)),
            pl.BlockSpec((1, 1, c_dim), lambda c, i: (c, 0, 0)),
            pl.BlockSpec((1, c_dim, chid), lambda c, i: (c, 0, 0)),
        ],
        out_specs=[
            pl.BlockSpec((1, _BS, c_dim), lambda c, i: (c, i, 0)),
            pl.BlockSpec((1, _BS, chid), lambda c, i: (c, i, 0)),
        ],
        out_shape=[
            jax.ShapeDtypeStruct((two, n, c_dim), jnp.float32),
            jax.ShapeDtypeStruct((two, n, chid), jnp.float32),
        ],
    )(s_all, y_all, deg2d, stats, b_all, gnw, gnb, gnms, w2_all)


def _tc_apply2(s_all, y2_all, deg2d, stats, b_all, gnw, gnb, gnms, h1_all):
    two, n, c_dim = y2_all.shape
    nb = n // _BS

    def body(s_ref, y_ref, d_ref, st_ref, b_ref, gw_ref, gb_ref, gm_ref, h1_ref,
             o_ref):
        i = pl.program_id(1)
        dinv = lax.rsqrt(d_ref[0, :, 0:1])
        pre = dinv * (s_ref[0] + y_ref[0]) + b_ref[0]
        g = _gn_apply(pre, st_ref, gw_ref, gb_ref, gm_ref, n)
        h2 = jnp.maximum(g + h1_ref[0], 0.0)

        @pl.when(i == 0)
        def _():
            o_ref[...] = jnp.zeros_like(o_ref)

        o_ref[0, 0:1, :] += jnp.sum(h2, axis=0, keepdims=True)
        o_ref[0, 1:2, :] = jnp.maximum(o_ref[0, 1:2, :],
                                       jnp.max(h2, axis=0, keepdims=True))

    return pl.pallas_call(
        body,
        grid=(two, nb),
        in_specs=[
            pl.BlockSpec((1, _BS, c_dim), lambda c, i: (c, i, 0)),
            pl.BlockSpec((1, _BS, c_dim), lambda c, i: (c, i, 0)),
            pl.BlockSpec((1, _BS, 16), lambda c, i: (c, i, 0)),
            pl.BlockSpec((1, 8, c_dim), lambda c, i: (c, 0, 0)),
            pl.BlockSpec((1, 1, c_dim), lambda c, i: (c, 0, 0)),
            pl.BlockSpec((1, 1, c_dim), lambda c, i: (c, 0, 0)),
            pl.BlockSpec((1, 1, c_dim), lambda c, i: (c, 0, 0)),
            pl.BlockSpec((1, 1, c_dim), lambda c, i: (c, 0, 0)),
            pl.BlockSpec((1, _BS, c_dim), lambda c, i: (c, i, 0)),
        ],
        out_specs=pl.BlockSpec((1, 8, c_dim), lambda c, i: (c, 0, 0)),
        out_shape=jax.ShapeDtypeStruct((two, 8, c_dim), jnp.float32),
    )(s_all, y2_all, deg2d, stats, b_all, gnw, gnb, gnms, h1_all)


def _tc_heads(pool, n, w1_all, b1_all, lnw_all, lnb_all, w2_all, b2_all,
              att_w, att_b):
    c_out = w2_all.shape[2]

    def body(pool_ref, w1_ref, b1_ref, lnw_ref, lnb_ref, w2_ref, b2_ref,
             aw_ref, ab_ref, po_ref, go_ref, awo_ref, comb_ref):
        outs = []
        for r in range(2):
            f = jnp.concatenate(
                [pool_ref[r, 0:1, :] * (1.0 / n), pool_ref[r, 1:2, :]], axis=1)
            h = jnp.dot(f, w1_ref[r], preferred_element_type=jnp.float32) + b1_ref[r]
            m = jnp.mean(h, axis=-1, keepdims=True)
            v = jnp.mean((h - m) * (h - m), axis=-1, keepdims=True)
            h = (h - m) / jnp.sqrt(v + EPS) * lnw_ref[r] + lnb_ref[r]
            h = jnp.maximum(h, 0.0)
            outs.append(jnp.dot(h, w2_ref[r], preferred_element_type=jnp.float32)
                        + b2_ref[r])
        po, go = outs
        comb = jnp.concatenate([po, go], axis=1)
        logits = jnp.dot(comb, aw_ref[...], preferred_element_type=jnp.float32) \
            + ab_ref[...]
        emax = jnp.max(logits, axis=1, keepdims=True)
        ex = jnp.exp(logits - emax)
        aw = ex / jnp.sum(ex, axis=1, keepdims=True)
        po_ref[...] = po
        go_ref[...] = go
        awo_ref[...] = aw
        comb_ref[...] = aw[:, 0:1] * po + aw[:, 1:2] * go

    return pl.pallas_call(
        body,
        out_shape=[
            jax.ShapeDtypeStruct((1, c_out), jnp.float32),
            jax.ShapeDtypeStruct((1, c_out), jnp.float32),
            jax.ShapeDtypeStruct((1, 2), jnp.float32),
            jax.ShapeDtypeStruct((1, c_out), jnp.float32),
        ],
    )(pool, w1_all, b1_all, lnw_all, lnb_all, w2_all, b2_all, att_w, att_b)


def kernel(x_protein, x_gene, ei_protein, ei_gene, params):
    p = params
    n, cin = x_protein.shape

    x_all = jnp.stack([x_protein, x_gene])
    w1_all = jnp.stack([p['W1p'], p['W1g']])
    w2_all = jnp.stack([p['W2p'], p['W2g']])
    b1_all = jnp.stack([p['b1p'], p['b1g']])[:, None, :]
    b2_all = jnp.stack([p['b2p'], p['b2g']])[:, None, :]
    gn1w = jnp.stack([p['gn1p_w'], p['gn1g_w']])[:, None, :]
    gn1b = jnp.stack([p['gn1p_b'], p['gn1g_b']])[:, None, :]
    gn1m = jnp.stack([p['gn1p_ms'], p['gn1g_ms']])[:, None, :]
    gn2w = jnp.stack([p['gn2p_w'], p['gn2g_w']])[:, None, :]
    gn2b = jnp.stack([p['gn2p_b'], p['gn2g_b']])[:, None, :]
    gn2m = jnp.stack([p['gn2p_ms'], p['gn2g_ms']])[:, None, :]
    hw1 = jnp.stack([p['hp_W1'], p['hg_W1']])
    hb1 = jnp.stack([p['hp_b1'], p['hg_b1']])[:, None, :]
    hlnw = jnp.stack([p['hp_lnw'], p['hg_lnw']])[:, None, :]
    hlnb = jnp.stack([p['hp_lnb'], p['hg_lnb']])[:, None, :]
    hw2 = jnp.stack([p['hp_W2'], p['hg_W2']])
    hb2 = jnp.stack([p['hp_b2'], p['hg_b2']])[:, None, :]
    att_w = p['att_W']
    att_b = p['att_b'][None, :]

    e = ei_protein.shape[1]
    align = _NS * _CH * 8
    ep = -(-e // align) * align
    pad = ep - e
    pad_dst = n + (jnp.arange(pad, dtype=jnp.int32) % _JUNK)
    pad_src = jnp.arange(pad, dtype=jnp.int32) % n
    dst_all = jnp.concatenate([ei_protein[1], pad_dst, ei_gene[1], pad_dst])
    src_all = jnp.concatenate(
        [ei_protein[0], pad_src, ei_gene[0] + n, pad_src]).reshape(-1, _CH)

    zeros_nc = jnp.zeros((n + _JUNK, cin), jnp.float32)

    deg2d = _sc_degree(dst_all, n)
    y1 = _tc_pre(x_all, w1_all, deg2d)
    s1 = _sc_scatter(y1.reshape(2 * n, -1), src_all, dst_all, zeros_nc, n)
    st1 = _tc_stats(s1, y1, deg2d, b1_all)
    h1, y2 = _tc_apply1(s1, y1, deg2d, st1, b1_all, gn1w, gn1b, gn1m, w2_all)
    s2 = _sc_scatter(y2.reshape(2 * n, -1), src_all, dst_all, zeros_nc, n)
    st2 = _tc_stats(s2, y2, deg2d, b2_all)
    pool = _tc_apply2(s2, y2, deg2d, st2, b2_all, gn2w, gn2b, gn2m, h1)
    po, go, aw, comb = _tc_heads(pool, n, hw1, hb1, hlnw, hlnb, hw2, hb2,
                                 att_w, att_b)
    return po, go, aw, comb
